# baseline (speedup 1.0000x reference)
import functools

import jax
import jax.numpy as jnp
from jax import lax
from jax.experimental import pallas as pl
from jax.experimental.pallas import tpu as pltpu
from jax.experimental.pallas import tpu_sc as plsc

N = 10000
E = 320000
B = 320000
H = 128
T = 2
BN_EPS = 1e-5

NC = 2
NS = 16
NW = NC * NS
EW = E // NW
CW = 80
NCH = EW // CW
NR = 10240
RPT = NR // NS
NP = NR
DPT = NP // NS

def _sc_mesh():
    return plsc.VectorSubcoreMesh(core_axis_name="c", subcore_axis_name="s",
                                  num_cores=NC, num_subcores=NS)


@functools.cache
def _deg_kernel_fn():
    return functools.partial(
        pl.kernel,
        out_type=jax.ShapeDtypeStruct((NC * NP,), jnp.float32),
        mesh=_sc_mesh(),
        scratch_types=[
            pltpu.VMEM((NCH, CW), jnp.int32),
            pltpu.VMEM((CW,), jnp.float32),
            pltpu.VMEM((DPT,), jnp.float32),
            pltpu.VMEM_SHARED((NP,), jnp.float32),
        ],
    )(_deg_body)


def _deg_body(dst_hbm, out_hbm, dst_v, ones_v, half_v, acc):
    c = lax.axis_index("c")
    s = lax.axis_index("s")
    wid = c * NS + s
    pltpu.sync_copy(dst_hbm.at[wid], dst_v)
    half = jnp.full((16,), 0.5, jnp.float32)
    one = jnp.ones((16,), jnp.float32)

    def fill_half(i, carry):
        half_v[pl.ds(i * 16, 16)] = half
        return carry

    lax.fori_loop(0, DPT // 16, fill_half, 0)

    def fill_one(i, carry):
        ones_v[pl.ds(i * 16, 16)] = one
        return carry

    lax.fori_loop(0, CW // 16, fill_one, 0)
    pltpu.sync_copy(half_v, acc.at[pl.ds(s * DPT, DPT)])
    plsc.subcore_barrier()

    def body(j, carry):
        pltpu.sync_copy(ones_v, acc.at[dst_v.at[j]], add=True)
        return carry

    lax.fori_loop(0, NCH, body, 0)
    plsc.subcore_barrier()
    pltpu.sync_copy(acc.at[pl.ds(s * DPT, DPT)],
                    out_hbm.at[pl.ds(c * NP + s * DPT, DPT)])


@functools.cache
def _agg_kernel_fn():
    return functools.partial(
        pl.kernel,
        out_type=jax.ShapeDtypeStruct((NC * NR, H), jnp.float32),
        mesh=_sc_mesh(),
        scratch_types=[
            pltpu.VMEM((3, CW), jnp.int32),
            pltpu.VMEM((3, CW), jnp.int32),
            pltpu.VMEM((CW, H), jnp.float32),
            pltpu.VMEM((CW, H), jnp.float32),
            pltpu.VMEM((CW, H), jnp.float32),
            pltpu.VMEM_SHARED((NR, H), jnp.float32),
            pltpu.SemaphoreType.DMA,
            pltpu.SemaphoreType.DMA,
            pltpu.SemaphoreType.DMA,
            pltpu.SemaphoreType.DMA,
            pltpu.SemaphoreType.DMA,
            pltpu.SemaphoreType.DMA,
        ],
    )(_agg_body)


def _agg_body(t_hbm, src_hbm, dst_hbm, out_hbm, sidx, didx, *rest):
    rows = rest[0:3]
    acc = rest[3]
    si = rest[4:7]
    sr = rest[7:10]
    c = lax.axis_index("c")
    s = lax.axis_index("s")
    wid = c * NS + s
    base = wid * EW

    def idx_load(j, b):
        pltpu.async_copy(src_hbm.at[pl.ds(base + j * CW, CW)], sidx.at[b],
                         si[b])
        pltpu.async_copy(dst_hbm.at[pl.ds(base + j * CW, CW)], didx.at[b],
                         si[b])

    def idx_wait(j, b):
        pltpu.make_async_copy(src_hbm.at[pl.ds(base + j * CW, CW)],
                              sidx.at[b], si[b]).wait()
        pltpu.make_async_copy(dst_hbm.at[pl.ds(base + j * CW, CW)],
                              didx.at[b], si[b]).wait()

    def gather(b):
        pltpu.async_copy(t_hbm.at[sidx.at[b]], rows[b], sr[b])

    def gather_wait(b):
        pltpu.make_async_copy(t_hbm.at[sidx.at[b]], rows[b], sr[b]).wait()

    idx_load(0, 0)
    idx_load(1, 1)
    idx_wait(0, 0)
    gather(0)
    idx_wait(1, 1)
    gather(1)

    r0 = s * RPT
    pltpu.sync_copy(t_hbm.at[pl.ds(r0, RPT)], acc.at[pl.ds(r0, RPT)])
    plsc.subcore_barrier()

    def visit(j, b, issue):
        gather_wait(b)
        if issue:
            b2 = (b + 2) % 3
            idx_load(j + 2, b2)
            idx_wait(j + 2, b2)
            gather(b2)
        pltpu.sync_copy(rows[b], acc.at[didx.at[b]], add=True)

    def body(g, carry):
        for b in range(3):
            visit(3 * g + b, b, True)
        return carry

    lax.fori_loop(0, NCH // 3, body, 0)
    for j in range(NCH - NCH % 3, NCH):
        visit(j, j % 3, False)
    plsc.subcore_barrier()
    pltpu.sync_copy(acc.at[pl.ds(r0, RPT)], out_hbm.at[pl.ds(c * NR + r0, RPT)])


@functools.cache
def _pair_kernel_fn(nch):
    bh = NW * nch * CW
    return functools.partial(
        pl.kernel,
        out_type=jax.ShapeDtypeStruct((bh, H), jnp.float32),
        mesh=_sc_mesh(),
        scratch_types=[
            pltpu.VMEM((nch, CW), jnp.int32),
            pltpu.VMEM((nch, CW), jnp.int32),
            pltpu.VMEM((4, CW), jnp.int32),
        ] + [pltpu.VMEM((CW, H), jnp.float32) for _ in range(4)]
          + [pltpu.VMEM_SHARED((NS * 4 * CW, H), jnp.float32)]
          + [pltpu.SemaphoreType.DMA for _ in range(6)],
    )(functools.partial(_pair_body, nch))


def _pair_body(nch, hs_hbm, ht_hbm, sidx_hbm, tidx_hbm, gsum_hbm,
               s_v, t_v, slotidx, *rest):
    bs = rest[0:2]
    bt = rest[2:4]
    shared = rest[4]
    ins = rest[5:7]
    outs = rest[7:11]
    c = lax.axis_index("c")
    s = lax.axis_index("s")
    wid = c * NS + s
    pltpu.sync_copy(sidx_hbm.at[wid], s_v)
    pltpu.sync_copy(tidx_hbm.at[wid], t_v)
    base = wid * (nch * CW)
    slot0 = s * (4 * CW)

    iota = lax.iota(jnp.int32, 16)
    for b in range(4):
        for k in range(CW // 16):
            slotidx[b, pl.ds(k * 16, 16)] = iota + (slot0 + b * CW + k * 16)

    def gather(j, p):
        pltpu.async_copy(hs_hbm.at[s_v.at[j]], bs[p], ins[p])
        pltpu.async_copy(ht_hbm.at[t_v.at[j]], bt[p], ins[p])

    def gather_wait(j, p):
        pltpu.make_async_copy(hs_hbm.at[s_v.at[j]], bs[p], ins[p]).wait()
        pltpu.make_async_copy(ht_hbm.at[t_v.at[j]], bt[p], ins[p]).wait()

    def write(j, b):
        pltpu.async_copy(shared.at[pl.ds(slot0 + b * CW, CW)],
                         gsum_hbm.at[pl.ds(base + j * CW, CW)], outs[b])

    def write_wait(j, b):
        pltpu.make_async_copy(shared.at[pl.ds(slot0 + b * CW, CW)],
                              gsum_hbm.at[pl.ds(base + j * CW, CW)],
                              outs[b]).wait()

    def visit(j, p, b, first, last):
        gather_wait(j, p)
        if not first:
            write_wait(j - 4, b)
        pltpu.sync_copy(bs[p], shared.at[pl.ds(slot0 + b * CW, CW)])
        pltpu.sync_copy(bt[p], shared.at[slotidx.at[b]], add=True)
        write(j, b)
        if not last:
            gather(j + 2, p)

    gather(0, 0)
    gather(1, 1)
    for j in range(4):
        visit(j, j % 2, j, True, False)

    ng = (nch - 8) // 4

    def body(g, carry):
        for b in range(4):
            j = 4 * g + 4 + b
            visit(j, b % 2, b, False, False)
        return carry

    lax.fori_loop(0, ng, body, 0)
    for j in range(4 * ng + 4, nch):
        visit(j, j % 2, j % 4, False, j + 2 >= nch)
    for j in range(nch - 4, nch):
        write_wait(j, j % 4)


BN = 1280
BPRED = 2560


def _embed_body(x_ref, deg_ref, wemb_ref, bemb_ref, wg0_ref, o_ref):
    h = jnp.maximum(
        jnp.dot(x_ref[...], wemb_ref[...], preferred_element_type=jnp.float32)
        + bemb_ref[...], 0.0)
    dinv = lax.rsqrt(deg_ref[...])
    o_ref[...] = jnp.dot(h, wg0_ref[...],
                         preferred_element_type=jnp.float32) * dinv


def _trans_body(p_ref, t_ref, deg_ref, sc_ref, w_ref, o_ref):
    dinv = lax.rsqrt(deg_ref[...])
    agg = p_ref[0] + p_ref[1] - t_ref[...]
    y = (dinv * agg) * sc_ref[0:1, :] + sc_ref[1:2, :]
    h = jnp.maximum(y, 0.0)
    o_ref[...] = jnp.dot(h, w_ref[...],
                         preferred_element_type=jnp.float32) * dinv


def _final_body(p_ref, t_ref, deg_ref, sc_ref, wa_ref, wb_ref, hs_ref, ht_ref):
    dinv = lax.rsqrt(deg_ref[...])
    y = (dinv * (p_ref[0] + p_ref[1] - t_ref[...])) * sc_ref[0:1, :] + sc_ref[1:2, :]
    hs_ref[...] = jnp.dot(y, wa_ref[...], preferred_element_type=jnp.float32)
    ht_ref[...] = jnp.dot(y, wb_ref[...], preferred_element_type=jnp.float32)


def _pred_body(gsum_ref, tf_ref, w1t_ref, b1_ref, w2_ref, b2_ref,
               w3_ref, b3_ref, o_ref):
    cterm = lax.dot_general(tf_ref[...], w1t_ref[...],
                            (((0,), (0,)), ((), ())),
                            preferred_element_type=jnp.float32)
    z = jnp.maximum(gsum_ref[...] + cterm + b1_ref[...], 0.0)
    z2 = jnp.maximum(
        jnp.dot(z, w2_ref[...], preferred_element_type=jnp.float32)
        + b2_ref[...], 0.0)
    row = lax.dot_general(w3_ref[...], z2, (((1,), (1,)), ((), ())),
                          preferred_element_type=jnp.float32) + b3_ref[...]
    for r in range(BPRED // 128):
        o_ref[0, pl.ds(r, 1), :] = row[:, r * 128:(r + 1) * 128]


def _full(shape):
    return pl.BlockSpec(shape, lambda i: (0,) * len(shape))


def _embed_call(x, deg, wemb, bemb, wg0):
    return pl.pallas_call(
        _embed_body,
        grid=(NR // BN,),
        in_specs=[
            pl.BlockSpec((BN, H), lambda i: (i, 0)),
            pl.BlockSpec((BN, 1), lambda i: (i, 0)),
            _full((H, H)),
            _full((1, H)),
            _full((H, H)),
        ],
        out_specs=pl.BlockSpec((BN, H), lambda i: (i, 0)),
        out_shape=jax.ShapeDtypeStruct((NR, H), jnp.float32),
    )(x, deg, wemb, bemb, wg0)


def _trans_call(p, t, deg, sc, w):
    return pl.pallas_call(
        _trans_body,
        grid=(NR // BN,),
        in_specs=[
            pl.BlockSpec((2, BN, H), lambda i: (0, i, 0)),
            pl.BlockSpec((BN, H), lambda i: (i, 0)),
            pl.BlockSpec((BN, 1), lambda i: (i, 0)),
            _full((2, H)),
            _full((H, H)),
        ],
        out_specs=pl.BlockSpec((BN, H), lambda i: (i, 0)),
        out_shape=jax.ShapeDtypeStruct((NR, H), jnp.float32),
    )(p, t, deg, sc, w)


def _final_call(p, t, deg, sc, wa, wb):
    return pl.pallas_call(
        _final_body,
        grid=(NR // BN,),
        in_specs=[
            pl.BlockSpec((2, BN, H), lambda i: (0, i, 0)),
            pl.BlockSpec((BN, H), lambda i: (i, 0)),
            pl.BlockSpec((BN, 1), lambda i: (i, 0)),
            _full((2, H)),
            _full((H, H)),
            _full((H, H)),
        ],
        out_specs=[
            pl.BlockSpec((BN, H), lambda i: (i, 0)),
            pl.BlockSpec((BN, H), lambda i: (i, 0)),
        ],
        out_shape=[
            jax.ShapeDtypeStruct((NR, H), jnp.float32),
            jax.ShapeDtypeStruct((NR, H), jnp.float32),
        ],
    )(p, t, deg, sc, wa, wb)


def _pred_call(gsum, tft, w1t, b1, w2, b2, w3, b3):
    bh = gsum.shape[0]
    return pl.pallas_call(
        _pred_body,
        grid=(bh // BPRED,),
        in_specs=[
            pl.BlockSpec((BPRED, H), lambda i: (i, 0)),
            pl.BlockSpec((T, BPRED), lambda i: (0, i)),
            _full((T, H)),
            _full((1, H)),
            _full((H, H // 2)),
            _full((1, H // 2)),
            _full((1, H // 2)),
            _full((1, 1)),
        ],
        out_specs=pl.BlockSpec((1, BPRED // 128, 128), lambda i: (i, 0, 0)),
        out_shape=jax.ShapeDtypeStruct((bh // BPRED, BPRED // 128, 128),
                                       jnp.float32),
    )(gsum, tft, w1t, b1, w2, b2, w3, b3)


def kernel(x, edge_index, source_nodes, target_nodes, time_feats,
           W_emb, b_emb,
           W_g0, b_g0, bn_gamma0, bn_beta0,
           W_g1, b_g1, bn_gamma1, bn_beta1,
           W_g2, b_g2, bn_gamma2, bn_beta2,
           Wp1, bp1, Wp2, bp2, Wp3, bp3):
    src = edge_index[0]
    dst = edge_index[1]
    dst3 = dst.reshape(NW, NCH, CW)
    nch_a, nch_b = 63, 62
    ba = NW * nch_a * CW
    sidx_a = source_nodes[:ba].reshape(NW, nch_a, CW)
    tidx_a = target_nodes[:ba].reshape(NW, nch_a, CW)
    sidx_b = source_nodes[ba:].reshape(NW, nch_b, CW)
    tidx_b = target_nodes[ba:].reshape(NW, nch_b, CW)

    xp = jnp.pad(x, ((0, NR - N), (0, 0)))
    degp = _deg_kernel_fn()(dst3)
    deg = (degp[:NP] + degp[NP:]).reshape(NR, 1)

    bscale = 1.0 / jnp.sqrt(jnp.float32(1.0 + BN_EPS))
    svec = [bn_gamma0 * bscale, bn_gamma1 * bscale, bn_gamma2 * bscale]
    cvec = [b_g0 * svec[0] + bn_beta0,
            b_g1 * svec[1] + bn_beta1,
            b_g2 * svec[2] + bn_beta2]
    sc0, sc1, sc2 = (jnp.stack([svec[i], cvec[i]]) for i in range(3))

    t0 = _embed_call(xp, deg, W_emb, b_emb.reshape(1, H), W_g0)
    p0 = _agg_kernel_fn()(t0, src, dst).reshape(2, NR, H)
    t1 = _trans_call(p0, t0, deg, sc0, W_g1)
    p1 = _agg_kernel_fn()(t1, src, dst).reshape(2, NR, H)
    t2 = _trans_call(p1, t1, deg, sc1, W_g2)
    p2 = _agg_kernel_fn()(t2, src, dst).reshape(2, NR, H)

    hs_tab, ht_tab = _final_call(p2, t2, deg, sc2, Wp1[:H], Wp1[H:2 * H])

    tft = time_feats.T
    w1t = Wp1[2 * H:]
    b1 = bp1.reshape(1, H)
    b2 = bp2.reshape(1, H // 2)
    w3 = Wp3.reshape(1, H // 2)
    b3 = bp3.reshape(1, 1)

    gsum_a = _pair_kernel_fn(nch_a)(hs_tab, ht_tab, sidx_a, tidx_a)
    gsum_b = _pair_kernel_fn(nch_b)(hs_tab, ht_tab, sidx_b, tidx_b)
    out_a = _pred_call(gsum_a, tft[:, :ba], w1t, b1, Wp2, b2, w3, b3)
    out_b = _pred_call(gsum_b, tft[:, ba:], w1t, b1, Wp2, b2, w3, b3)
    return jnp.concatenate([out_a.reshape(ba), out_b.reshape(B - ba)])


# speedup vs baseline: 1.0031x; 1.0031x over previous
import functools

import jax
import jax.numpy as jnp
from jax import lax
from jax.experimental import pallas as pl
from jax.experimental.pallas import tpu as pltpu
from jax.experimental.pallas import tpu_sc as plsc

N = 10000
E = 320000
B = 320000
H = 128
T = 2
BN_EPS = 1e-5

NC = 2
NS = 16
NW = NC * NS
EW = E // NW
CW = 80
NCH = EW // CW
NR = 10240
RPT = NR // NS
NP = NR
DPT = NP // NS

def _sc_mesh():
    return plsc.VectorSubcoreMesh(core_axis_name="c", subcore_axis_name="s",
                                  num_cores=NC, num_subcores=NS)


@functools.cache
def _deg_kernel_fn():
    return functools.partial(
        pl.kernel,
        out_type=jax.ShapeDtypeStruct((NC * NP,), jnp.float32),
        mesh=_sc_mesh(),
        scratch_types=[
            pltpu.VMEM((NCH, CW), jnp.int32),
            pltpu.VMEM((CW,), jnp.float32),
            pltpu.VMEM((DPT,), jnp.float32),
            pltpu.VMEM_SHARED((NP,), jnp.float32),
        ],
    )(_deg_body)


def _deg_body(dst_hbm, out_hbm, dst_v, ones_v, half_v, acc):
    c = lax.axis_index("c")
    s = lax.axis_index("s")
    wid = c * NS + s
    pltpu.sync_copy(dst_hbm.at[wid], dst_v)
    half = jnp.full((16,), 0.5, jnp.float32)
    one = jnp.ones((16,), jnp.float32)

    def fill_half(i, carry):
        half_v[pl.ds(i * 16, 16)] = half
        return carry

    lax.fori_loop(0, DPT // 16, fill_half, 0)

    def fill_one(i, carry):
        ones_v[pl.ds(i * 16, 16)] = one
        return carry

    lax.fori_loop(0, CW // 16, fill_one, 0)
    pltpu.sync_copy(half_v, acc.at[pl.ds(s * DPT, DPT)])
    plsc.subcore_barrier()

    def body(j, carry):
        pltpu.sync_copy(ones_v, acc.at[dst_v.at[j]], add=True)
        return carry

    lax.fori_loop(0, NCH, body, 0)
    plsc.subcore_barrier()
    pltpu.sync_copy(acc.at[pl.ds(s * DPT, DPT)],
                    out_hbm.at[pl.ds(c * NP + s * DPT, DPT)])


@functools.cache
def _agg_kernel_fn():
    return functools.partial(
        pl.kernel,
        out_type=jax.ShapeDtypeStruct((NC * NR, H), jnp.float32),
        mesh=_sc_mesh(),
        scratch_types=[
            pltpu.VMEM((3, CW), jnp.int32),
            pltpu.VMEM((3, CW), jnp.int32),
            pltpu.VMEM((CW, H), jnp.float32),
            pltpu.VMEM((CW, H), jnp.float32),
            pltpu.VMEM((CW, H), jnp.float32),
            pltpu.VMEM_SHARED((NR, H), jnp.float32),
            pltpu.SemaphoreType.DMA,
            pltpu.SemaphoreType.DMA,
            pltpu.SemaphoreType.DMA,
            pltpu.SemaphoreType.DMA,
            pltpu.SemaphoreType.DMA,
            pltpu.SemaphoreType.DMA,
        ],
    )(_agg_body)


def _agg_body(t_hbm, src_hbm, dst_hbm, out_hbm, sidx, didx, *rest):
    rows = rest[0:3]
    acc = rest[3]
    si = rest[4:7]
    sr = rest[7:10]
    c = lax.axis_index("c")
    s = lax.axis_index("s")
    wid = c * NS + s
    base = wid * EW

    def idx_load(j, b):
        pltpu.async_copy(src_hbm.at[pl.ds(base + j * CW, CW)], sidx.at[b],
                         si[b])
        pltpu.async_copy(dst_hbm.at[pl.ds(base + j * CW, CW)], didx.at[b],
                         si[b])

    def idx_wait(j, b):
        pltpu.make_async_copy(src_hbm.at[pl.ds(base + j * CW, CW)],
                              sidx.at[b], si[b]).wait()
        pltpu.make_async_copy(dst_hbm.at[pl.ds(base + j * CW, CW)],
                              didx.at[b], si[b]).wait()

    def gather(b):
        pltpu.async_copy(t_hbm.at[sidx.at[b]], rows[b], sr[b])

    def gather_wait(b):
        pltpu.make_async_copy(t_hbm.at[sidx.at[b]], rows[b], sr[b]).wait()

    idx_load(0, 0)
    idx_load(1, 1)
    idx_load(2, 2)
    idx_wait(0, 0)
    gather(0)
    idx_wait(1, 1)
    gather(1)

    r0 = s * RPT
    pltpu.sync_copy(t_hbm.at[pl.ds(r0, RPT)], acc.at[pl.ds(r0, RPT)])
    plsc.subcore_barrier()

    def visit(j, b, do_gather, do_load):
        gather_wait(b)
        if do_gather:
            b2 = (b + 2) % 3
            idx_wait(j + 2, b2)
            gather(b2)
        pltpu.sync_copy(rows[b], acc.at[didx.at[b]], add=True)
        if do_load:
            idx_load(j + 3, b)

    def body(g, carry):
        for b in range(3):
            visit(3 * g + b, b, True, True)
        return carry

    nfull = NCH // 3 - 2
    lax.fori_loop(0, nfull, body, 0)
    for j in range(3 * nfull, NCH):
        visit(j, j % 3, j + 2 < NCH, j + 3 < NCH)
    plsc.subcore_barrier()
    pltpu.sync_copy(acc.at[pl.ds(r0, RPT)], out_hbm.at[pl.ds(c * NR + r0, RPT)])


@functools.cache
def _pair_kernel_fn(nch):
    bh = NW * nch * CW
    return functools.partial(
        pl.kernel,
        out_type=jax.ShapeDtypeStruct((bh, H), jnp.float32),
        mesh=_sc_mesh(),
        scratch_types=[
            pltpu.VMEM((nch, CW), jnp.int32),
            pltpu.VMEM((nch, CW), jnp.int32),
            pltpu.VMEM((4, CW), jnp.int32),
        ] + [pltpu.VMEM((CW, H), jnp.float32) for _ in range(4)]
          + [pltpu.VMEM_SHARED((NS * 4 * CW, H), jnp.float32)]
          + [pltpu.SemaphoreType.DMA for _ in range(6)],
    )(functools.partial(_pair_body, nch))


def _pair_body(nch, hs_hbm, ht_hbm, sidx_hbm, tidx_hbm, gsum_hbm,
               s_v, t_v, slotidx, *rest):
    bs = rest[0:2]
    bt = rest[2:4]
    shared = rest[4]
    ins = rest[5:7]
    outs = rest[7:11]
    c = lax.axis_index("c")
    s = lax.axis_index("s")
    wid = c * NS + s
    pltpu.sync_copy(sidx_hbm.at[wid], s_v)
    pltpu.sync_copy(tidx_hbm.at[wid], t_v)
    base = wid * (nch * CW)
    slot0 = s * (4 * CW)

    iota = lax.iota(jnp.int32, 16)
    for b in range(4):
        for k in range(CW // 16):
            slotidx[b, pl.ds(k * 16, 16)] = iota + (slot0 + b * CW + k * 16)

    def gather(j, p):
        pltpu.async_copy(hs_hbm.at[s_v.at[j]], bs[p], ins[p])
        pltpu.async_copy(ht_hbm.at[t_v.at[j]], bt[p], ins[p])

    def gather_wait(j, p):
        pltpu.make_async_copy(hs_hbm.at[s_v.at[j]], bs[p], ins[p]).wait()
        pltpu.make_async_copy(ht_hbm.at[t_v.at[j]], bt[p], ins[p]).wait()

    def write(j, b):
        pltpu.async_copy(shared.at[pl.ds(slot0 + b * CW, CW)],
                         gsum_hbm.at[pl.ds(base + j * CW, CW)], outs[b])

    def write_wait(j, b):
        pltpu.make_async_copy(shared.at[pl.ds(slot0 + b * CW, CW)],
                              gsum_hbm.at[pl.ds(base + j * CW, CW)],
                              outs[b]).wait()

    def visit(j, p, b, first, last):
        gather_wait(j, p)
        if not first:
            write_wait(j - 4, b)
        pltpu.sync_copy(bs[p], shared.at[pl.ds(slot0 + b * CW, CW)])
        pltpu.sync_copy(bt[p], shared.at[slotidx.at[b]], add=True)
        write(j, b)
        if not last:
            gather(j + 2, p)

    gather(0, 0)
    gather(1, 1)
    for j in range(4):
        visit(j, j % 2, j, True, False)

    ng = (nch - 8) // 4

    def body(g, carry):
        for b in range(4):
            j = 4 * g + 4 + b
            visit(j, b % 2, b, False, False)
        return carry

    lax.fori_loop(0, ng, body, 0)
    for j in range(4 * ng + 4, nch):
        visit(j, j % 2, j % 4, False, j + 2 >= nch)
    for j in range(nch - 4, nch):
        write_wait(j, j % 4)


BN = 1280
BPRED = 2560


def _embed_body(x_ref, deg_ref, wemb_ref, bemb_ref, wg0_ref, o_ref):
    h = jnp.maximum(
        jnp.dot(x_ref[...], wemb_ref[...], preferred_element_type=jnp.float32)
        + bemb_ref[...], 0.0)
    dinv = lax.rsqrt(deg_ref[...])
    o_ref[...] = jnp.dot(h, wg0_ref[...],
                         preferred_element_type=jnp.float32) * dinv


def _trans_body(p_ref, t_ref, deg_ref, sc_ref, w_ref, o_ref):
    dinv = lax.rsqrt(deg_ref[...])
    agg = p_ref[0] + p_ref[1] - t_ref[...]
    y = (dinv * agg) * sc_ref[0:1, :] + sc_ref[1:2, :]
    h = jnp.maximum(y, 0.0)
    o_ref[...] = jnp.dot(h, w_ref[...],
                         preferred_element_type=jnp.float32) * dinv


def _final_body(p_ref, t_ref, deg_ref, sc_ref, wa_ref, wb_ref, hs_ref, ht_ref):
    dinv = lax.rsqrt(deg_ref[...])
    y = (dinv * (p_ref[0] + p_ref[1] - t_ref[...])) * sc_ref[0:1, :] + sc_ref[1:2, :]
    hs_ref[...] = jnp.dot(y, wa_ref[...], preferred_element_type=jnp.float32)
    ht_ref[...] = jnp.dot(y, wb_ref[...], preferred_element_type=jnp.float32)


def _pred_body(gsum_ref, tf_ref, w1t_ref, b1_ref, w2_ref, b2_ref,
               w3_ref, b3_ref, o_ref):
    cterm = lax.dot_general(tf_ref[...], w1t_ref[...],
                            (((0,), (0,)), ((), ())),
                            preferred_element_type=jnp.float32)
    z = jnp.maximum(gsum_ref[...] + cterm + b1_ref[...], 0.0)
    z2 = jnp.maximum(
        jnp.dot(z, w2_ref[...], preferred_element_type=jnp.float32)
        + b2_ref[...], 0.0)
    row = lax.dot_general(w3_ref[...], z2, (((1,), (1,)), ((), ())),
                          preferred_element_type=jnp.float32) + b3_ref[...]
    for r in range(BPRED // 128):
        o_ref[0, pl.ds(r, 1), :] = row[:, r * 128:(r + 1) * 128]


def _full(shape):
    return pl.BlockSpec(shape, lambda i: (0,) * len(shape))


def _embed_call(x, deg, wemb, bemb, wg0):
    return pl.pallas_call(
        _embed_body,
        grid=(NR // BN,),
        in_specs=[
            pl.BlockSpec((BN, H), lambda i: (i, 0)),
            pl.BlockSpec((BN, 1), lambda i: (i, 0)),
            _full((H, H)),
            _full((1, H)),
            _full((H, H)),
        ],
        out_specs=pl.BlockSpec((BN, H), lambda i: (i, 0)),
        out_shape=jax.ShapeDtypeStruct((NR, H), jnp.float32),
    )(x, deg, wemb, bemb, wg0)


def _trans_call(p, t, deg, sc, w):
    return pl.pallas_call(
        _trans_body,
        grid=(NR // BN,),
        in_specs=[
            pl.BlockSpec((2, BN, H), lambda i: (0, i, 0)),
            pl.BlockSpec((BN, H), lambda i: (i, 0)),
            pl.BlockSpec((BN, 1), lambda i: (i, 0)),
            _full((2, H)),
            _full((H, H)),
        ],
        out_specs=pl.BlockSpec((BN, H), lambda i: (i, 0)),
        out_shape=jax.ShapeDtypeStruct((NR, H), jnp.float32),
    )(p, t, deg, sc, w)


def _final_call(p, t, deg, sc, wa, wb):
    return pl.pallas_call(
        _final_body,
        grid=(NR // BN,),
        in_specs=[
            pl.BlockSpec((2, BN, H), lambda i: (0, i, 0)),
            pl.BlockSpec((BN, H), lambda i: (i, 0)),
            pl.BlockSpec((BN, 1), lambda i: (i, 0)),
            _full((2, H)),
            _full((H, H)),
            _full((H, H)),
        ],
        out_specs=[
            pl.BlockSpec((BN, H), lambda i: (i, 0)),
            pl.BlockSpec((BN, H), lambda i: (i, 0)),
        ],
        out_shape=[
            jax.ShapeDtypeStruct((NR, H), jnp.float32),
            jax.ShapeDtypeStruct((NR, H), jnp.float32),
        ],
    )(p, t, deg, sc, wa, wb)


def _pred_call(gsum, tft, w1t, b1, w2, b2, w3, b3):
    bh = gsum.shape[0]
    return pl.pallas_call(
        _pred_body,
        grid=(bh // BPRED,),
        in_specs=[
            pl.BlockSpec((BPRED, H), lambda i: (i, 0)),
            pl.BlockSpec((T, BPRED), lambda i: (0, i)),
            _full((T, H)),
            _full((1, H)),
            _full((H, H // 2)),
            _full((1, H // 2)),
            _full((1, H // 2)),
            _full((1, 1)),
        ],
        out_specs=pl.BlockSpec((1, BPRED // 128, 128), lambda i: (i, 0, 0)),
        out_shape=jax.ShapeDtypeStruct((bh // BPRED, BPRED // 128, 128),
                                       jnp.float32),
    )(gsum, tft, w1t, b1, w2, b2, w3, b3)


def kernel(x, edge_index, source_nodes, target_nodes, time_feats,
           W_emb, b_emb,
           W_g0, b_g0, bn_gamma0, bn_beta0,
           W_g1, b_g1, bn_gamma1, bn_beta1,
           W_g2, b_g2, bn_gamma2, bn_beta2,
           Wp1, bp1, Wp2, bp2, Wp3, bp3):
    src = edge_index[0]
    dst = edge_index[1]
    dst3 = dst.reshape(NW, NCH, CW)
    nch_a, nch_b = 63, 62
    ba = NW * nch_a * CW
    sidx_a = source_nodes[:ba].reshape(NW, nch_a, CW)
    tidx_a = target_nodes[:ba].reshape(NW, nch_a, CW)
    sidx_b = source_nodes[ba:].reshape(NW, nch_b, CW)
    tidx_b = target_nodes[ba:].reshape(NW, nch_b, CW)

    xp = jnp.pad(x, ((0, NR - N), (0, 0)))
    degp = _deg_kernel_fn()(dst3)
    deg = (degp[:NP] + degp[NP:]).reshape(NR, 1)

    bscale = 1.0 / jnp.sqrt(jnp.float32(1.0 + BN_EPS))
    svec = [bn_gamma0 * bscale, bn_gamma1 * bscale, bn_gamma2 * bscale]
    cvec = [b_g0 * svec[0] + bn_beta0,
            b_g1 * svec[1] + bn_beta1,
            b_g2 * svec[2] + bn_beta2]
    sc0, sc1, sc2 = (jnp.stack([svec[i], cvec[i]]) for i in range(3))

    t0 = _embed_call(xp, deg, W_emb, b_emb.reshape(1, H), W_g0)
    p0 = _agg_kernel_fn()(t0, src, dst).reshape(2, NR, H)
    t1 = _trans_call(p0, t0, deg, sc0, W_g1)
    p1 = _agg_kernel_fn()(t1, src, dst).reshape(2, NR, H)
    t2 = _trans_call(p1, t1, deg, sc1, W_g2)
    p2 = _agg_kernel_fn()(t2, src, dst).reshape(2, NR, H)

    hs_tab, ht_tab = _final_call(p2, t2, deg, sc2, Wp1[:H], Wp1[H:2 * H])

    tft = time_feats.T
    w1t = Wp1[2 * H:]
    b1 = bp1.reshape(1, H)
    b2 = bp2.reshape(1, H // 2)
    w3 = Wp3.reshape(1, H // 2)
    b3 = bp3.reshape(1, 1)

    gsum_a = _pair_kernel_fn(nch_a)(hs_tab, ht_tab, sidx_a, tidx_a)
    gsum_b = _pair_kernel_fn(nch_b)(hs_tab, ht_tab, sidx_b, tidx_b)
    out_a = _pred_call(gsum_a, tft[:, :ba], w1t, b1, Wp2, b2, w3, b3)
    out_b = _pred_call(gsum_b, tft[:, ba:], w1t, b1, Wp2, b2, w3, b3)
    return jnp.concatenate([out_a.reshape(ba), out_b.reshape(B - ba)])


# speedup vs baseline: 1.0502x; 1.0469x over previous
import functools

import jax
import jax.numpy as jnp
from jax import lax
from jax.experimental import pallas as pl
from jax.experimental.pallas import tpu as pltpu
from jax.experimental.pallas import tpu_sc as plsc

N = 10000
E = 320000
B = 320000
H = 128
T = 2
BN_EPS = 1e-5

NC = 2
NS = 16
NW = NC * NS
EW = E // NW
CW = 80
NCH = EW // CW
NR = 10240
RPT = NR // NS
NP = NR
DPT = NP // NS

def _sc_mesh():
    return plsc.VectorSubcoreMesh(core_axis_name="c", subcore_axis_name="s",
                                  num_cores=NC, num_subcores=NS)


@functools.cache
def _deg_kernel_fn():
    return functools.partial(
        pl.kernel,
        out_type=jax.ShapeDtypeStruct((NC * NP,), jnp.float32),
        mesh=_sc_mesh(),
        scratch_types=[
            pltpu.VMEM((NCH, CW), jnp.int32),
            pltpu.VMEM((CW,), jnp.float32),
            pltpu.VMEM((DPT,), jnp.float32),
            pltpu.VMEM_SHARED((NP,), jnp.float32),
        ],
    )(_deg_body)


def _deg_body(dst_hbm, out_hbm, dst_v, ones_v, half_v, acc):
    c = lax.axis_index("c")
    s = lax.axis_index("s")
    wid = c * NS + s
    pltpu.sync_copy(dst_hbm.at[wid], dst_v)
    half = jnp.full((16,), 0.5, jnp.float32)
    one = jnp.ones((16,), jnp.float32)

    def fill_half(i, carry):
        half_v[pl.ds(i * 16, 16)] = half
        return carry

    lax.fori_loop(0, DPT // 16, fill_half, 0)

    def fill_one(i, carry):
        ones_v[pl.ds(i * 16, 16)] = one
        return carry

    lax.fori_loop(0, CW // 16, fill_one, 0)
    pltpu.sync_copy(half_v, acc.at[pl.ds(s * DPT, DPT)])
    plsc.subcore_barrier()

    def body(j, carry):
        pltpu.sync_copy(ones_v, acc.at[dst_v.at[j]], add=True)
        return carry

    lax.fori_loop(0, NCH, body, 0)
    plsc.subcore_barrier()
    pltpu.sync_copy(acc.at[pl.ds(s * DPT, DPT)],
                    out_hbm.at[pl.ds(c * NP + s * DPT, DPT)])


@functools.cache
def _agg_kernel_fn():
    return functools.partial(
        pl.kernel,
        out_type=jax.ShapeDtypeStruct((NC * NR, H), jnp.float32),
        mesh=_sc_mesh(),
        scratch_types=[
            pltpu.VMEM((2, CW), jnp.int32),
            pltpu.VMEM((NCH, CW), jnp.int32),
            pltpu.VMEM((CW, H), jnp.float32),
            pltpu.VMEM((CW, H), jnp.float32),
            pltpu.VMEM_SHARED((NR, H), jnp.float32),
            pltpu.SemaphoreType.DMA,
            pltpu.SemaphoreType.DMA,
            pltpu.SemaphoreType.DMA,
            pltpu.SemaphoreType.DMA,
        ],
    )(_agg_body)


def _agg_body(t_hbm, src_hbm, dst_hbm, out_hbm, sidx, dst_v, rows0, rows1,
              acc, si0, si1, sr0, sr1):
    c = lax.axis_index("c")
    s = lax.axis_index("s")
    wid = c * NS + s
    base = wid * EW
    pltpu.sync_copy(dst_hbm.at[wid], dst_v)
    r0 = s * RPT
    pltpu.sync_copy(t_hbm.at[pl.ds(r0, RPT)], acc.at[pl.ds(r0, RPT)])
    plsc.subcore_barrier()

    def idx_load(j, b, sem):
        pltpu.async_copy(src_hbm.at[pl.ds(base + j * CW, CW)], sidx.at[b], sem)

    def idx_wait(j, b, sem):
        pltpu.make_async_copy(src_hbm.at[pl.ds(base + j * CW, CW)],
                              sidx.at[b], sem).wait()

    def gather(b, rows, sem):
        pltpu.async_copy(t_hbm.at[sidx.at[b]], rows, sem)

    def gather_wait(rows, sem):
        pltpu.make_async_copy(t_hbm.at[sidx.at[0]], rows, sem).wait()

    idx_load(0, 0, si0)
    idx_load(1, 1, si1)
    idx_wait(0, 0, si0)
    gather(0, rows0, sr0)
    idx_wait(1, 1, si1)
    gather(1, rows1, sr1)

    def body(g, carry):
        j0 = 2 * g
        j1 = j0 + 1
        gather_wait(rows0, sr0)
        idx_load(j0 + 2, 0, si0)
        pltpu.sync_copy(rows0, acc.at[dst_v.at[j0]], add=True)
        idx_wait(j0 + 2, 0, si0)
        gather(0, rows0, sr0)
        gather_wait(rows1, sr1)

        @pl.when(g < NCH // 2 - 1)
        def _():
            idx_load(j1 + 2, 1, si1)

        pltpu.sync_copy(rows1, acc.at[dst_v.at[j1]], add=True)

        @pl.when(g < NCH // 2 - 1)
        def _():
            idx_wait(j1 + 2, 1, si1)
            gather(1, rows1, sr1)

        return carry

    lax.fori_loop(0, NCH // 2, body, 0)
    gather_wait(rows0, sr0)
    pltpu.sync_copy(rows0, acc.at[dst_v.at[NCH - 1]], add=True)
    plsc.subcore_barrier()
    pltpu.sync_copy(acc.at[pl.ds(r0, RPT)], out_hbm.at[pl.ds(c * NR + r0, RPT)])


@functools.cache
def _pair_kernel_fn(nch):
    bh = NW * nch * CW
    return functools.partial(
        pl.kernel,
        out_type=jax.ShapeDtypeStruct((bh, H), jnp.float32),
        mesh=_sc_mesh(),
        scratch_types=[
            pltpu.VMEM((nch, CW), jnp.int32),
            pltpu.VMEM((nch, CW), jnp.int32),
            pltpu.VMEM((4, CW), jnp.int32),
        ] + [pltpu.VMEM((CW, H), jnp.float32) for _ in range(4)]
          + [pltpu.VMEM_SHARED((NS * 4 * CW, H), jnp.float32)]
          + [pltpu.SemaphoreType.DMA for _ in range(6)],
    )(functools.partial(_pair_body, nch))


def _pair_body(nch, hs_hbm, ht_hbm, sidx_hbm, tidx_hbm, gsum_hbm,
               s_v, t_v, slotidx, *rest):
    bs = rest[0:2]
    bt = rest[2:4]
    shared = rest[4]
    ins = rest[5:7]
    outs = rest[7:11]
    c = lax.axis_index("c")
    s = lax.axis_index("s")
    wid = c * NS + s
    pltpu.sync_copy(sidx_hbm.at[wid], s_v)
    pltpu.sync_copy(tidx_hbm.at[wid], t_v)
    base = wid * (nch * CW)
    slot0 = s * (4 * CW)

    iota = lax.iota(jnp.int32, 16)
    for b in range(4):
        for k in range(CW // 16):
            slotidx[b, pl.ds(k * 16, 16)] = iota + (slot0 + b * CW + k * 16)

    def gather(j, p):
        pltpu.async_copy(hs_hbm.at[s_v.at[j]], bs[p], ins[p])
        pltpu.async_copy(ht_hbm.at[t_v.at[j]], bt[p], ins[p])

    def gather_wait(j, p):
        pltpu.make_async_copy(hs_hbm.at[s_v.at[j]], bs[p], ins[p]).wait()
        pltpu.make_async_copy(ht_hbm.at[t_v.at[j]], bt[p], ins[p]).wait()

    def write(j, b):
        pltpu.async_copy(shared.at[pl.ds(slot0 + b * CW, CW)],
                         gsum_hbm.at[pl.ds(base + j * CW, CW)], outs[b])

    def write_wait(j, b):
        pltpu.make_async_copy(shared.at[pl.ds(slot0 + b * CW, CW)],
                              gsum_hbm.at[pl.ds(base + j * CW, CW)],
                              outs[b]).wait()

    def visit(j, p, b, first, last):
        gather_wait(j, p)
        if not first:
            write_wait(j - 4, b)
        pltpu.sync_copy(bs[p], shared.at[pl.ds(slot0 + b * CW, CW)])
        pltpu.sync_copy(bt[p], shared.at[slotidx.at[b]], add=True)
        write(j, b)
        if not last:
            gather(j + 2, p)

    gather(0, 0)
    gather(1, 1)
    for j in range(4):
        visit(j, j % 2, j, True, False)

    ng = (nch - 8) // 4

    def body(g, carry):
        for b in range(4):
            j = 4 * g + 4 + b
            visit(j, b % 2, b, False, False)
        return carry

    lax.fori_loop(0, ng, body, 0)
    for j in range(4 * ng + 4, nch):
        visit(j, j % 2, j % 4, False, j + 2 >= nch)
    for j in range(nch - 4, nch):
        write_wait(j, j % 4)


BN = 1280
BPRED = 2560


def _embed_body(x_ref, deg_ref, wemb_ref, bemb_ref, wg0_ref, o_ref):
    h = jnp.maximum(
        jnp.dot(x_ref[...], wemb_ref[...], preferred_element_type=jnp.float32)
        + bemb_ref[...], 0.0)
    dinv = lax.rsqrt(deg_ref[...])
    o_ref[...] = jnp.dot(h, wg0_ref[...],
                         preferred_element_type=jnp.float32) * dinv


def _trans_body(p_ref, t_ref, deg_ref, sc_ref, w_ref, o_ref):
    dinv = lax.rsqrt(deg_ref[...])
    agg = p_ref[0] + p_ref[1] - t_ref[...]
    y = (dinv * agg) * sc_ref[0:1, :] + sc_ref[1:2, :]
    h = jnp.maximum(y, 0.0)
    o_ref[...] = jnp.dot(h, w_ref[...],
                         preferred_element_type=jnp.float32) * dinv


def _final_body(p_ref, t_ref, deg_ref, sc_ref, wa_ref, wb_ref, hs_ref, ht_ref):
    dinv = lax.rsqrt(deg_ref[...])
    y = (dinv * (p_ref[0] + p_ref[1] - t_ref[...])) * sc_ref[0:1, :] + sc_ref[1:2, :]
    hs_ref[...] = jnp.dot(y, wa_ref[...], preferred_element_type=jnp.float32)
    ht_ref[...] = jnp.dot(y, wb_ref[...], preferred_element_type=jnp.float32)


def _pred_body(gsum_ref, tf_ref, w1t_ref, b1_ref, w2_ref, b2_ref,
               w3_ref, b3_ref, o_ref):
    cterm = lax.dot_general(tf_ref[...], w1t_ref[...],
                            (((0,), (0,)), ((), ())),
                            preferred_element_type=jnp.float32)
    z = jnp.maximum(gsum_ref[...] + cterm + b1_ref[...], 0.0)
    z2 = jnp.maximum(
        jnp.dot(z, w2_ref[...], preferred_element_type=jnp.float32)
        + b2_ref[...], 0.0)
    row = lax.dot_general(w3_ref[...], z2, (((1,), (1,)), ((), ())),
                          preferred_element_type=jnp.float32) + b3_ref[...]
    for r in range(BPRED // 128):
        o_ref[0, pl.ds(r, 1), :] = row[:, r * 128:(r + 1) * 128]


def _full(shape):
    return pl.BlockSpec(shape, lambda i: (0,) * len(shape))


def _embed_call(x, deg, wemb, bemb, wg0):
    return pl.pallas_call(
        _embed_body,
        grid=(NR // BN,),
        in_specs=[
            pl.BlockSpec((BN, H), lambda i: (i, 0)),
            pl.BlockSpec((BN, 1), lambda i: (i, 0)),
            _full((H, H)),
            _full((1, H)),
            _full((H, H)),
        ],
        out_specs=pl.BlockSpec((BN, H), lambda i: (i, 0)),
        out_shape=jax.ShapeDtypeStruct((NR, H), jnp.float32),
    )(x, deg, wemb, bemb, wg0)


def _trans_call(p, t, deg, sc, w):
    return pl.pallas_call(
        _trans_body,
        grid=(NR // BN,),
        in_specs=[
            pl.BlockSpec((2, BN, H), lambda i: (0, i, 0)),
            pl.BlockSpec((BN, H), lambda i: (i, 0)),
            pl.BlockSpec((BN, 1), lambda i: (i, 0)),
            _full((2, H)),
            _full((H, H)),
        ],
        out_specs=pl.BlockSpec((BN, H), lambda i: (i, 0)),
        out_shape=jax.ShapeDtypeStruct((NR, H), jnp.float32),
    )(p, t, deg, sc, w)


def _final_call(p, t, deg, sc, wa, wb):
    return pl.pallas_call(
        _final_body,
        grid=(NR // BN,),
        in_specs=[
            pl.BlockSpec((2, BN, H), lambda i: (0, i, 0)),
            pl.BlockSpec((BN, H), lambda i: (i, 0)),
            pl.BlockSpec((BN, 1), lambda i: (i, 0)),
            _full((2, H)),
            _full((H, H)),
            _full((H, H)),
        ],
        out_specs=[
            pl.BlockSpec((BN, H), lambda i: (i, 0)),
            pl.BlockSpec((BN, H), lambda i: (i, 0)),
        ],
        out_shape=[
            jax.ShapeDtypeStruct((NR, H), jnp.float32),
            jax.ShapeDtypeStruct((NR, H), jnp.float32),
        ],
    )(p, t, deg, sc, wa, wb)


def _pred_call(gsum, tft, w1t, b1, w2, b2, w3, b3):
    bh = gsum.shape[0]
    return pl.pallas_call(
        _pred_body,
        grid=(bh // BPRED,),
        in_specs=[
            pl.BlockSpec((BPRED, H), lambda i: (i, 0)),
            pl.BlockSpec((T, BPRED), lambda i: (0, i)),
            _full((T, H)),
            _full((1, H)),
            _full((H, H // 2)),
            _full((1, H // 2)),
            _full((1, H // 2)),
            _full((1, 1)),
        ],
        out_specs=pl.BlockSpec((1, BPRED // 128, 128), lambda i: (i, 0, 0)),
        out_shape=jax.ShapeDtypeStruct((bh // BPRED, BPRED // 128, 128),
                                       jnp.float32),
    )(gsum, tft, w1t, b1, w2, b2, w3, b3)


def kernel(x, edge_index, source_nodes, target_nodes, time_feats,
           W_emb, b_emb,
           W_g0, b_g0, bn_gamma0, bn_beta0,
           W_g1, b_g1, bn_gamma1, bn_beta1,
           W_g2, b_g2, bn_gamma2, bn_beta2,
           Wp1, bp1, Wp2, bp2, Wp3, bp3):
    src = edge_index[0]
    dst = edge_index[1]
    dst3 = dst.reshape(NW, NCH, CW)
    nch_a, nch_b = 63, 62
    ba = NW * nch_a * CW
    sidx_a = source_nodes[:ba].reshape(NW, nch_a, CW)
    tidx_a = target_nodes[:ba].reshape(NW, nch_a, CW)
    sidx_b = source_nodes[ba:].reshape(NW, nch_b, CW)
    tidx_b = target_nodes[ba:].reshape(NW, nch_b, CW)

    xp = jnp.pad(x, ((0, NR - N), (0, 0)))
    degp = _deg_kernel_fn()(dst3)
    deg = (degp[:NP] + degp[NP:]).reshape(NR, 1)

    bscale = 1.0 / jnp.sqrt(jnp.float32(1.0 + BN_EPS))
    svec = [bn_gamma0 * bscale, bn_gamma1 * bscale, bn_gamma2 * bscale]
    cvec = [b_g0 * svec[0] + bn_beta0,
            b_g1 * svec[1] + bn_beta1,
            b_g2 * svec[2] + bn_beta2]
    sc0, sc1, sc2 = (jnp.stack([svec[i], cvec[i]]) for i in range(3))

    t0 = _embed_call(xp, deg, W_emb, b_emb.reshape(1, H), W_g0)
    p0 = _agg_kernel_fn()(t0, src, dst3).reshape(2, NR, H)
    t1 = _trans_call(p0, t0, deg, sc0, W_g1)
    p1 = _agg_kernel_fn()(t1, src, dst3).reshape(2, NR, H)
    t2 = _trans_call(p1, t1, deg, sc1, W_g2)
    p2 = _agg_kernel_fn()(t2, src, dst3).reshape(2, NR, H)

    hs_tab, ht_tab = _final_call(p2, t2, deg, sc2, Wp1[:H], Wp1[H:2 * H])

    tft = time_feats.T
    w1t = Wp1[2 * H:]
    b1 = bp1.reshape(1, H)
    b2 = bp2.reshape(1, H // 2)
    w3 = Wp3.reshape(1, H // 2)
    b3 = bp3.reshape(1, 1)

    gsum_a = _pair_kernel_fn(nch_a)(hs_tab, ht_tab, sidx_a, tidx_a)
    gsum_b = _pair_kernel_fn(nch_b)(hs_tab, ht_tab, sidx_b, tidx_b)
    out_a = _pred_call(gsum_a, tft[:, :ba], w1t, b1, Wp2, b2, w3, b3)
    out_b = _pred_call(gsum_b, tft[:, ba:], w1t, b1, Wp2, b2, w3, b3)
    return jnp.concatenate([out_a.reshape(ba), out_b.reshape(B - ba)])
